# baseline (speedup 1.0000x reference)
import functools

import jax
import jax.numpy as jnp
import numpy as np
from jax import lax
from jax.experimental import pallas as pl
from jax.experimental.pallas import tpu as pltpu
from jax.experimental.pallas import tpu_sc as plsc

K = 8
EPS = float(np.finfo(np.float32).tiny)
L = 16
N = 4096
NG = N // L
R = 512
NC = 2
NS = 16
NW = NC * NS
RPW = R // NW


def _make_sc_topk():
    mesh = plsc.VectorSubcoreMesh(core_axis_name="c", subcore_axis_name="s")

    @functools.partial(
        pl.kernel,
        mesh=mesh,
        compiler_params=pltpu.CompilerParams(needs_layout_passes=False),
        out_type=(
            jax.ShapeDtypeStruct((R, N), jnp.float32),
            jax.ShapeDtypeStruct((R, K, N), jnp.float32),
        ),
        scratch_types=[
            pltpu.VMEM((N,), jnp.float32),
            pltpu.VMEM((N,), jnp.float32),
            pltpu.VMEM((N,), jnp.float32),
            pltpu.VMEM((K, N), jnp.float32),
            pltpu.SemaphoreType.DMA,
        ],
    )
    def sc_topk(x_hbm, khot_hbm, km_hbm, xv, tv, kv, bv, sem_out):
        c = lax.axis_index("c")
        s = lax.axis_index("s")
        wid = s * NC + c

        def drain_prev_row():
            for _ in range(K + 1):
                pltpu.make_async_copy(kv, khot_hbm.at[0], sem_out).wait()

        def row_body(r, carry):
            row = wid * RPW + r
            pltpu.sync_copy(x_hbm.at[row], xv)

            def max_g(j, acc):
                return jnp.maximum(acc, xv[pl.ds(j * L, L)])

            mx = lax.fori_loop(
                0, NG, max_g, jnp.full((L,), -jnp.inf, jnp.float32))
            m = jnp.max(mx)

            def exp_g(j, acc):
                v = jnp.exp(xv[pl.ds(j * L, L)] - m)
                tv[pl.ds(j * L, L)] = v
                return acc + v

            part = lax.fori_loop(0, NG, exp_g, jnp.zeros((L,), jnp.float32))
            q = jnp.sum(part)

            @pl.when(r != 0)
            def _():
                drain_prev_row()

            for i in range(K):
                last = i == K - 1
                inv = jnp.ones((L,), jnp.float32) / q

                def g(j, acc, i=i, inv=inv, last=last):
                    sl = pl.ds(j * L, L)
                    p = tv[sl] * inv
                    bv[i, sl] = p
                    if last:
                        return acc
                    t2 = p * jnp.maximum(1.0 - p, EPS)
                    tv[sl] = t2
                    return acc + t2

                part = lax.fori_loop(
                    0, NG, g, jnp.zeros((L,), jnp.float32))
                pltpu.async_copy(bv.at[i], km_hbm.at[row, i], sem_out)
                q = jnp.sum(part)

            def kh_g(j, carry2):
                sl = pl.ds(j * L, L)
                acc = bv[0, sl]
                for i in range(1, K):
                    acc = acc + bv[i, sl]
                kv[sl] = acc
                return carry2

            lax.fori_loop(0, NG, kh_g, 0)
            pltpu.async_copy(kv, khot_hbm.at[row], sem_out)
            return carry

        lax.fori_loop(0, RPW, row_body, 0)
        drain_prev_row()

    return sc_topk


_SC_TOPK = _make_sc_topk()


def kernel(scores):
    b, h, s, n = scores.shape
    khot, km = _SC_TOPK(scores.reshape(R, N))
    return khot.reshape(b, h, s, n), km.reshape(b, h, s, K, n)


# speedup vs baseline: 2.7253x; 2.7253x over previous
import functools

import jax
import jax.numpy as jnp
import numpy as np
from jax import lax
from jax.experimental import pallas as pl
from jax.experimental.pallas import tpu as pltpu
from jax.experimental.pallas import tpu_sc as plsc

K = 8
EPS = float(np.finfo(np.float32).tiny)
L = 16
N = 4096
NG = N // L
U = 4
R = 512
NC = 2
NS = 16
NW = NC * NS
RPW = R // NW


def _make_sc_topk():
    mesh = plsc.VectorSubcoreMesh(core_axis_name="c", subcore_axis_name="s")

    @functools.partial(
        pl.kernel,
        mesh=mesh,
        compiler_params=pltpu.CompilerParams(needs_layout_passes=False),
        out_type=(
            jax.ShapeDtypeStruct((R, N), jnp.float32),
            jax.ShapeDtypeStruct((R, K, N), jnp.float32),
        ),
        scratch_types=[
            pltpu.VMEM((N,), jnp.float32),
            pltpu.VMEM((N,), jnp.float32),
            pltpu.VMEM((N,), jnp.float32),
            pltpu.VMEM((K, N), jnp.float32),
            pltpu.SemaphoreType.DMA,
        ],
    )
    def sc_topk(x_hbm, khot_hbm, km_hbm, xv, tv, kv, bv, sem_out):
        c = lax.axis_index("c")
        s = lax.axis_index("s")
        wid = s * NC + c

        zeros = jnp.zeros((L,), jnp.float32)
        ones = jnp.ones((L,), jnp.float32)

        def drain_prev_row():
            for _ in range(K + 1):
                pltpu.make_async_copy(kv, khot_hbm.at[0], sem_out).wait()

        def row_body(r, carry):
            row = wid * RPW + r
            pltpu.sync_copy(x_hbm.at[row], xv)

            @plsc.parallel_loop(0, NG, step=U, carry=(zeros,) * U)
            def max_body(j, accs):
                return tuple(
                    jnp.maximum(accs[k], xv[pl.ds((j + k) * L, L)])
                    for k in range(U)
                )

            m = jnp.max(jnp.maximum(jnp.maximum(max_body[0], max_body[1]),
                                    jnp.maximum(max_body[2], max_body[3])))

            @plsc.parallel_loop(0, NG, step=U, carry=(zeros,) * U)
            def exp_body(j, accs):
                out = []
                for k in range(U):
                    sl = pl.ds((j + k) * L, L)
                    v = jnp.exp(xv[sl] - m)
                    tv[sl] = v
                    out.append(accs[k] + v)
                return tuple(out)

            q = jnp.sum(exp_body[0] + exp_body[1] + exp_body[2] + exp_body[3])

            @pl.when(r != 0)
            def _():
                drain_prev_row()

            for i in range(K):
                first, last = i == 0, i == K - 1
                inv = ones / q

                @plsc.parallel_loop(0, NG, step=U, carry=(zeros,) * U)
                def iter_body(j, accs, i=i, inv=inv, first=first, last=last):
                    out = []
                    for k in range(U):
                        sl = pl.ds((j + k) * L, L)
                        p = tv[sl] * inv
                        bv[i, sl] = p
                        if first:
                            kv[sl] = p
                        else:
                            plsc.addupdate(kv.at[sl], p)
                        if last:
                            out.append(accs[k])
                        else:
                            t2 = p * jnp.maximum(1.0 - p, EPS)
                            tv[sl] = t2
                            out.append(accs[k] + t2)
                    return tuple(out)

                pltpu.async_copy(bv.at[i], km_hbm.at[row, i], sem_out)
                q = jnp.sum(iter_body[0] + iter_body[1]
                            + iter_body[2] + iter_body[3])

            pltpu.async_copy(kv, khot_hbm.at[row], sem_out)
            return carry

        lax.fori_loop(0, RPW, row_body, 0)
        drain_prev_row()

    return sc_topk


_SC_TOPK = _make_sc_topk()


def kernel(scores):
    b, h, s, n = scores.shape
    khot, km = _SC_TOPK(scores.reshape(R, N))
    return khot.reshape(b, h, s, n), km.reshape(b, h, s, K, n)


# speedup vs baseline: 2.7368x; 1.0042x over previous
import functools

import jax
import jax.numpy as jnp
import numpy as np
from jax import lax
from jax.experimental import pallas as pl
from jax.experimental.pallas import tpu as pltpu
from jax.experimental.pallas import tpu_sc as plsc

K = 8
EPS = float(np.finfo(np.float32).tiny)
L = 16
N = 4096
NG = N // L
U = 8
R = 512
NC = 2
NS = 16
NW = NC * NS
RPW = R // NW


def _make_sc_topk():
    mesh = plsc.VectorSubcoreMesh(core_axis_name="c", subcore_axis_name="s")

    @functools.partial(
        pl.kernel,
        mesh=mesh,
        compiler_params=pltpu.CompilerParams(needs_layout_passes=False),
        out_type=(
            jax.ShapeDtypeStruct((R, N), jnp.float32),
            jax.ShapeDtypeStruct((R, K, N), jnp.float32),
        ),
        scratch_types=[
            pltpu.VMEM((N,), jnp.float32),
            pltpu.VMEM((N,), jnp.float32),
            pltpu.VMEM((N,), jnp.float32),
            pltpu.VMEM((K, N), jnp.float32),
            pltpu.SemaphoreType.DMA,
        ],
    )
    def sc_topk(x_hbm, khot_hbm, km_hbm, xv, tv, kv, bv, sem_out):
        c = lax.axis_index("c")
        s = lax.axis_index("s")
        wid = s * NC + c

        zeros = jnp.zeros((L,), jnp.float32)
        ones = jnp.ones((L,), jnp.float32)

        def drain_prev_row():
            for _ in range(K + 1):
                pltpu.make_async_copy(kv, khot_hbm.at[0], sem_out).wait()

        def row_body(r, carry):
            row = wid * RPW + r
            pltpu.sync_copy(x_hbm.at[row], xv)

            @plsc.parallel_loop(0, NG, step=U, carry=(zeros,) * U)
            def max_body(j, accs):
                return tuple(
                    jnp.maximum(accs[k], xv[pl.ds((j + k) * L, L)])
                    for k in range(U)
                )

            mm = max_body
            while len(mm) > 1:
                mm = tuple(jnp.maximum(mm[2 * a], mm[2 * a + 1])
                           for a in range(len(mm) // 2))
            m = jnp.max(mm[0])

            @plsc.parallel_loop(0, NG, step=U, carry=(zeros,) * U)
            def exp_body(j, accs):
                out = []
                for k in range(U):
                    sl = pl.ds((j + k) * L, L)
                    v = jnp.exp(xv[sl] - m)
                    tv[sl] = v
                    out.append(accs[k] + v)
                return tuple(out)

            q = jnp.sum(sum(exp_body[1:], exp_body[0]))

            @pl.when(r != 0)
            def _():
                drain_prev_row()

            for i in range(K):
                first, last = i == 0, i == K - 1
                inv = ones / q

                @plsc.parallel_loop(0, NG, step=U, carry=(zeros,) * U)
                def iter_body(j, accs, i=i, inv=inv, first=first, last=last):
                    out = []
                    for k in range(U):
                        sl = pl.ds((j + k) * L, L)
                        p = tv[sl] * inv
                        bv[i, sl] = p
                        if first:
                            kv[sl] = p
                        else:
                            plsc.addupdate(kv.at[sl], p)
                        if last:
                            out.append(accs[k])
                        else:
                            t2 = p * jnp.maximum(1.0 - p, EPS)
                            tv[sl] = t2
                            out.append(accs[k] + t2)
                    return tuple(out)

                pltpu.async_copy(bv.at[i], km_hbm.at[row, i], sem_out)
                q = jnp.sum(sum(iter_body[1:], iter_body[0]))

            pltpu.async_copy(kv, khot_hbm.at[row], sem_out)
            return carry

        lax.fori_loop(0, RPW, row_body, 0)
        drain_prev_row()

    return sc_topk


_SC_TOPK = _make_sc_topk()


def kernel(scores):
    b, h, s, n = scores.shape
    khot, km = _SC_TOPK(scores.reshape(R, N))
    return khot.reshape(b, h, s, n), km.reshape(b, h, s, K, n)


# speedup vs baseline: 3.3926x; 1.2396x over previous
import functools

import jax
import jax.numpy as jnp
import numpy as np
from jax import lax
from jax.experimental import pallas as pl
from jax.experimental.pallas import tpu as pltpu
from jax.experimental.pallas import tpu_sc as plsc

K = 8
EPS = float(np.finfo(np.float32).tiny)
L = 16
N = 4096
NG = N // L
U = 8
R = 512
NC = 2
NS = 16
NW = NC * NS
RPW = R // NW


def _make_sc_slices():
    mesh = plsc.VectorSubcoreMesh(core_axis_name="c", subcore_axis_name="s")

    @functools.partial(
        pl.kernel,
        mesh=mesh,
        compiler_params=pltpu.CompilerParams(needs_layout_passes=False),
        out_type=jax.ShapeDtypeStruct((R, K, N), jnp.float32),
        scratch_types=[
            pltpu.VMEM((N,), jnp.float32),
            pltpu.VMEM((N,), jnp.float32),
            pltpu.VMEM((K, N), jnp.float32),
            pltpu.SemaphoreType.DMA,
        ],
    )
    def sc_slices(x_hbm, km_hbm, xv, tv, bv, sem_out):
        c = lax.axis_index("c")
        s = lax.axis_index("s")
        wid = s * NC + c

        zeros = jnp.zeros((L,), jnp.float32)
        ones = jnp.ones((L,), jnp.float32)

        def drain_prev_row():
            for _ in range(K):
                pltpu.make_async_copy(xv, km_hbm.at[0, 0], sem_out).wait()

        def row_body(r, carry):
            row = wid * RPW + r
            pltpu.sync_copy(x_hbm.at[row], xv)

            @plsc.parallel_loop(0, NG, step=U, carry=(zeros,) * U)
            def max_body(j, accs):
                return tuple(
                    jnp.maximum(accs[k], xv[pl.ds((j + k) * L, L)])
                    for k in range(U)
                )

            mm = max_body
            while len(mm) > 1:
                mm = tuple(jnp.maximum(mm[2 * a], mm[2 * a + 1])
                           for a in range(len(mm) // 2))
            m = jnp.max(mm[0])

            @plsc.parallel_loop(0, NG, step=U, carry=(zeros,) * U)
            def exp_body(j, accs):
                out = []
                for k in range(U):
                    sl = pl.ds((j + k) * L, L)
                    v = jnp.exp(xv[sl] - m)
                    tv[sl] = v
                    out.append(accs[k] + v)
                return tuple(out)

            q = jnp.sum(sum(exp_body[1:], exp_body[0]))

            @pl.when(r != 0)
            def _():
                drain_prev_row()

            for i in range(K):
                last = i == K - 1
                inv = ones / q

                @plsc.parallel_loop(0, NG, step=U, carry=(zeros,) * U)
                def iter_body(j, accs, i=i, inv=inv, last=last):
                    out = []
                    for k in range(U):
                        sl = pl.ds((j + k) * L, L)
                        p = tv[sl] * inv
                        bv[i, sl] = p
                        if last:
                            out.append(accs[k])
                        else:
                            t2 = p * jnp.maximum(1.0 - p, EPS)
                            tv[sl] = t2
                            out.append(accs[k] + t2)
                    return tuple(out)

                pltpu.async_copy(bv.at[i], km_hbm.at[row, i], sem_out)
                q = jnp.sum(sum(iter_body[1:], iter_body[0]))

            return carry

        lax.fori_loop(0, RPW, row_body, 0)
        drain_prev_row()

    return sc_slices


_SC_SLICES = _make_sc_slices()

_TC_BLK = 16


def _tc_khot_body(x_ref, kh_ref):
    x = x_ref[...]
    m = jnp.max(x, axis=-1, keepdims=True)
    t = jnp.exp(x - m)
    kh = jnp.zeros_like(t)
    for i in range(K):
        s = jnp.sum(t, axis=-1, keepdims=True)
        p = t * (1.0 / s)
        kh = kh + p
        if i < K - 1:
            t = t * jnp.maximum(1.0 - p, EPS)
    kh_ref[...] = kh


def _tc_khot(x2d):
    return pl.pallas_call(
        _tc_khot_body,
        grid=(R // _TC_BLK,),
        in_specs=[pl.BlockSpec((_TC_BLK, N), lambda i: (i, 0))],
        out_specs=pl.BlockSpec((_TC_BLK, N), lambda i: (i, 0)),
        out_shape=jax.ShapeDtypeStruct((R, N), jnp.float32),
    )(x2d)


def kernel(scores):
    b, h, s, n = scores.shape
    x2d = scores.reshape(R, N)
    km = _SC_SLICES(x2d)
    khot = _tc_khot(x2d)
    return khot.reshape(b, h, s, n), km.reshape(b, h, s, K, n)


# speedup vs baseline: 3.5767x; 1.0543x over previous
import functools

import jax
import jax.numpy as jnp
import numpy as np
from jax import lax
from jax.experimental import pallas as pl
from jax.experimental.pallas import tpu as pltpu
from jax.experimental.pallas import tpu_sc as plsc

K = 8
EPS = float(np.finfo(np.float32).tiny)
L = 16
N = 4096
NG = N // L
U = 8
R = 512
NC = 2
NS = 16
NW = NC * NS
RPW = R // NW


def _make_sc_slices():
    mesh = plsc.VectorSubcoreMesh(core_axis_name="c", subcore_axis_name="s")

    @functools.partial(
        pl.kernel,
        mesh=mesh,
        compiler_params=pltpu.CompilerParams(needs_layout_passes=False),
        out_type=jax.ShapeDtypeStruct((16, 8, 4, K, N), jnp.float32),
        scratch_types=[
            pltpu.VMEM((N,), jnp.float32),
            pltpu.VMEM((N,), jnp.float32),
            pltpu.VMEM((K, N), jnp.float32),
            pltpu.SemaphoreType.DMA,
        ],
    )
    def sc_slices(x_hbm, km_hbm, xv, tv, bv, sem_out):
        c = lax.axis_index("c")
        s = lax.axis_index("s")
        wid = s * NC + c

        zeros = jnp.zeros((L,), jnp.float32)
        ones = jnp.ones((L,), jnp.float32)

        def drain_prev_row():
            for _ in range(K):
                pltpu.make_async_copy(xv, km_hbm.at[0, 0, 0, 0], sem_out).wait()

        def row_body(r, carry):
            row = wid * RPW + r
            rb = row >> 5
            rh = (row >> 2) & 7
            rs = row & 3
            pltpu.sync_copy(x_hbm.at[rb, rh, rs], xv)

            @plsc.parallel_loop(0, NG, step=U, carry=(zeros,) * U)
            def max_body(j, accs):
                return tuple(
                    jnp.maximum(accs[k], xv[pl.ds((j + k) * L, L)])
                    for k in range(U)
                )

            mm = max_body
            while len(mm) > 1:
                mm = tuple(jnp.maximum(mm[2 * a], mm[2 * a + 1])
                           for a in range(len(mm) // 2))
            m = jnp.max(mm[0])

            @plsc.parallel_loop(0, NG, step=U, carry=(zeros,) * U)
            def exp_body(j, accs):
                out = []
                for k in range(U):
                    sl = pl.ds((j + k) * L, L)
                    v = jnp.exp(xv[sl] - m)
                    tv[sl] = v
                    out.append(accs[k] + v)
                return tuple(out)

            q = jnp.sum(sum(exp_body[1:], exp_body[0]))

            @pl.when(r != 0)
            def _():
                drain_prev_row()

            for i in range(K):
                last = i == K - 1
                inv = ones / q

                @plsc.parallel_loop(0, NG, step=U, carry=(zeros,) * U)
                def iter_body(j, accs, i=i, inv=inv, last=last):
                    out = []
                    for k in range(U):
                        sl = pl.ds((j + k) * L, L)
                        p = tv[sl] * inv
                        bv[i, sl] = p
                        if last:
                            out.append(accs[k])
                        else:
                            t2 = p * jnp.maximum(1.0 - p, EPS)
                            tv[sl] = t2
                            out.append(accs[k] + t2)
                    return tuple(out)

                pltpu.async_copy(bv.at[i], km_hbm.at[rb, rh, rs, i], sem_out)
                q = jnp.sum(sum(iter_body[1:], iter_body[0]))

            return carry

        lax.fori_loop(0, RPW, row_body, 0)
        drain_prev_row()

    return sc_slices


_SC_SLICES = _make_sc_slices()

def _tc_khot_body(x_ref, kh_ref):
    x = x_ref[...]
    m = jnp.max(x, axis=-1, keepdims=True)
    t = jnp.exp(x - m)
    kh = jnp.zeros_like(t)
    for i in range(K):
        s = jnp.sum(t, axis=-1, keepdims=True)
        p = t * (1.0 / s)
        kh = kh + p
        if i < K - 1:
            t = t * jnp.maximum(1.0 - p, EPS)
    kh_ref[...] = kh


def _tc_khot(scores):
    return pl.pallas_call(
        _tc_khot_body,
        grid=(16,),
        in_specs=[pl.BlockSpec((1, 8, 4, N), lambda i: (i, 0, 0, 0))],
        out_specs=pl.BlockSpec((1, 8, 4, N), lambda i: (i, 0, 0, 0)),
        out_shape=jax.ShapeDtypeStruct((16, 8, 4, N), jnp.float32),
    )(scores)


def kernel(scores):
    km = _SC_SLICES(scores)
    khot = _tc_khot(scores)
    return khot, km


# speedup vs baseline: 3.9683x; 1.1095x over previous
import functools

import jax
import jax.numpy as jnp
import numpy as np
from jax import lax
from jax.experimental import pallas as pl
from jax.experimental.pallas import tpu as pltpu
from jax.experimental.pallas import tpu_sc as plsc

K = 8
EPS = float(np.finfo(np.float32).tiny)
L = 16
N = 4096
NG = N // L
U = 8
R = 512
NC = 2
NS = 16
NW = NC * NS
RPW = R // NW


def _make_sc_slices():
    mesh = plsc.VectorSubcoreMesh(core_axis_name="c", subcore_axis_name="s")

    @functools.partial(
        pl.kernel,
        mesh=mesh,
        compiler_params=pltpu.CompilerParams(needs_layout_passes=False),
        out_type=jax.ShapeDtypeStruct((16, 8, 4, K, N), jnp.float32),
        scratch_types=[
            pltpu.VMEM((N,), jnp.float32),
            pltpu.VMEM((N,), jnp.float32),
            pltpu.VMEM((N,), jnp.float32),
            pltpu.VMEM((K, N), jnp.float32),
            pltpu.SemaphoreType.DMA,
            pltpu.SemaphoreType.DMA,
        ],
    )
    def sc_slices(x_hbm, km_hbm, xva, xvb, tv, bv, sem_in, sem_out):
        c = lax.axis_index("c")
        s = lax.axis_index("s")
        wid = s * NC + c

        zeros = jnp.zeros((L,), jnp.float32)
        ones = jnp.ones((L,), jnp.float32)

        def rbhs(row):
            return row >> 5, (row >> 2) & 7, row & 3

        def fetch(r, xv):
            rb, rh, rs = rbhs(wid * RPW + jnp.minimum(r, RPW - 1))
            pltpu.async_copy(x_hbm.at[rb, rh, rs], xv, sem_in)

        def wait_fetch(xv):
            pltpu.make_async_copy(x_hbm.at[0, 0, 0], xv, sem_in).wait()

        def drain_prev_row():
            for _ in range(K):
                pltpu.make_async_copy(tv, km_hbm.at[0, 0, 0, 0],
                                      sem_out).wait()

        def row_body(r, xv, xv_next):
            row = wid * RPW + r
            rb, rh, rs = rbhs(row)
            wait_fetch(xv)
            fetch(r + 1, xv_next)

            @plsc.parallel_loop(0, NG, step=U, carry=(zeros,) * U)
            def max_body(j, accs):
                return tuple(
                    jnp.maximum(accs[k], xv[pl.ds((j + k) * L, L)])
                    for k in range(U)
                )

            mm = max_body
            while len(mm) > 1:
                mm = tuple(jnp.maximum(mm[2 * a], mm[2 * a + 1])
                           for a in range(len(mm) // 2))
            m = jnp.max(mm[0])

            @plsc.parallel_loop(0, NG, step=U, carry=(zeros,) * U)
            def exp_body(j, accs):
                out = []
                for k in range(U):
                    sl = pl.ds((j + k) * L, L)
                    v = jnp.exp(xv[sl] - m)
                    tv[sl] = v
                    out.append(accs[k] + v)
                return tuple(out)

            q = jnp.sum(sum(exp_body[1:], exp_body[0]))

            @pl.when(r != 0)
            def _():
                drain_prev_row()

            for i in range(K):
                last = i == K - 1
                inv = ones / q

                @plsc.parallel_loop(0, NG, step=U, carry=(zeros,) * U)
                def iter_body(j, accs, i=i, inv=inv, last=last):
                    out = []
                    for k in range(U):
                        sl = pl.ds((j + k) * L, L)
                        p = tv[sl] * inv
                        bv[i, sl] = p
                        if last:
                            out.append(accs[k])
                        else:
                            t2 = p * jnp.maximum(1.0 - p, EPS)
                            tv[sl] = t2
                            out.append(accs[k] + t2)
                    return tuple(out)

                pltpu.async_copy(bv.at[i], km_hbm.at[rb, rh, rs, i], sem_out)
                q = jnp.sum(sum(iter_body[1:], iter_body[0]))

        fetch(0, xva)

        def pair_body(rr, carry):
            row_body(2 * rr, xva, xvb)
            row_body(2 * rr + 1, xvb, xva)
            return carry

        lax.fori_loop(0, RPW // 2, pair_body, 0)
        drain_prev_row()
        wait_fetch(xva)

    return sc_slices


_SC_SLICES = _make_sc_slices()


def _tc_khot_body(x_ref, kh_ref):
    x = x_ref[...]
    m = jnp.max(x, axis=-1, keepdims=True)
    t = jnp.exp(x - m)
    kh = jnp.zeros_like(t)
    for i in range(K):
        s = jnp.sum(t, axis=-1, keepdims=True)
        p = t * (1.0 / s)
        kh = kh + p
        if i < K - 1:
            t = t * jnp.maximum(1.0 - p, EPS)
    kh_ref[...] = kh


def _tc_khot(scores):
    return pl.pallas_call(
        _tc_khot_body,
        grid=(16,),
        in_specs=[pl.BlockSpec((1, 8, 4, N), lambda i: (i, 0, 0, 0))],
        out_specs=pl.BlockSpec((1, 8, 4, N), lambda i: (i, 0, 0, 0)),
        out_shape=jax.ShapeDtypeStruct((16, 8, 4, N), jnp.float32),
    )(scores)


def kernel(scores):
    km = _SC_SLICES(scores)
    khot = _tc_khot(scores)
    return khot, km
